# baseline (speedup 1.0000x reference)
import functools

import jax
import jax.numpy as jnp
from jax.experimental import pallas as pl
from jax.experimental.pallas import tpu as pltpu


_K = 8


def _f32dot(a, b):
    return jnp.dot(a, b, preferred_element_type=jnp.float32)


def _bf16dot(a, b):
    return _f32dot(a, b).astype(jnp.bfloat16)


def _scan_kernel(x_ref, whx_ref, bhx_ref, whh_ref, bhh_ref, woh_ref, h0_ref,
                 l_ref, z_ref, sprev_ref, hfin_ref,
                 wbf, wxp, bp, wohbf, wkp, carry, *, k_steps, n_chunks, b):
    j = pl.program_id(0)
    cb = n_chunks * b

    @pl.when(j == 0)
    def _prep():
        w = whh_ref[...].astype(jnp.bfloat16)
        wbf[...] = w
        wxp[...] = _bf16dot(whx_ref[...].astype(jnp.bfloat16), w)
        bp[...] = _f32dot(bhx_ref[...].astype(jnp.bfloat16), w) + bhh_ref[...]
        woh_bf = woh_ref[...].astype(jnp.bfloat16)
        wohbf[...] = woh_bf
        w2 = _bf16dot(w, w)
        w4 = _bf16dot(w2, w2)
        z1 = _bf16dot(w, woh_bf)
        z2 = _bf16dot(w2, woh_bf)
        z12 = jnp.concatenate([z1, z2], axis=1)
        z14 = jnp.concatenate([z12, _bf16dot(w2, z12)], axis=1)
        z18 = jnp.concatenate([z14, _bf16dot(w4, z14)], axis=1)
        if k_steps == 8:
            wkp[...] = _bf16dot(w4, w4)
            z_ref[...] = z18
        else:
            w8 = _bf16dot(w4, w4)
            wkp[...] = _bf16dot(w8, w8)
            z_ref[...] = jnp.concatenate([z18, _bf16dot(w8, z18)], axis=1)
        carry[...] = jnp.zeros_like(carry)
        carry[0:b, :] = h0_ref[...]

    @pl.when(j > 0)
    def _emit_l():
        l_ref[...] = _bf16dot(carry[...].astype(jnp.bfloat16),
                              wohbf[...]).reshape(l_ref.shape)

    @pl.when(j < k_steps)
    def _step():
        v = _f32dot(x_ref[...].reshape(cb, -1).astype(jnp.bfloat16),
                    wxp[...]) + bp[...]
        r = _f32dot(carry[...].astype(jnp.bfloat16), wbf[...]) + v
        carry[...] = r

    @pl.when(j == k_steps)
    def _boundary():
        wk = wkp[...]
        sprev_ref[0] = jnp.zeros_like(sprev_ref[0])
        s = carry[0:b, :]
        for c in range(1, n_chunks):
            sprev_ref[c] = s.astype(jnp.bfloat16)
            s = _f32dot(s.astype(jnp.bfloat16), wk) + carry[c * b:(c + 1) * b, :]
        hfin_ref[...] = s


def _head_kernel(l_ref, sprev_ref, z_ref, boh_ref, y_ref, *, cb):
    sp = sprev_ref[...].reshape(cb, -1)
    logits = (l_ref[...].reshape(cb, -1).astype(jnp.float32)
              + _f32dot(sp, z_ref[...]) + boh_ref[...])
    m = jnp.max(logits, axis=1, keepdims=True)
    sh = logits - m
    lse = jnp.log(jnp.sum(jnp.exp(sh), axis=1, keepdims=True))
    y_ref[...] = (sh - lse).reshape(y_ref.shape)


def kernel(xs, h0, whx, bhx, whh, bhh, woh, boh):
    T, B, I = xs.shape
    H = whh.shape[0]
    O = woh.shape[1]
    K = _K
    assert T % K == 0, (T, K)
    C = T // K
    CB = C * B

    xs4 = xs.reshape(C, K, B, I)
    l_all, z, s_prev, h_final = pl.pallas_call(
        functools.partial(_scan_kernel, k_steps=K, n_chunks=C, b=B),
        grid=(K + 1,),
        in_specs=[
            pl.BlockSpec((C, 1, B, I),
                         lambda j: (0, jnp.minimum(j, K - 1), 0, 0)),
            pl.BlockSpec((I, H), lambda j: (0, 0)),
            pl.BlockSpec((1, H), lambda j: (0, 0)),
            pl.BlockSpec((H, H), lambda j: (0, 0)),
            pl.BlockSpec((1, H), lambda j: (0, 0)),
            pl.BlockSpec((H, O), lambda j: (0, 0)),
            pl.BlockSpec((B, H), lambda j: (0, 0)),
        ],
        out_specs=(
            pl.BlockSpec((1, CB, O), lambda j: (jnp.maximum(j - 1, 0), 0, 0)),
            pl.BlockSpec((H, K * O), lambda j: (0, 0)),
            pl.BlockSpec((C, B, H), lambda j: (0, 0, 0)),
            pl.BlockSpec((B, H), lambda j: (0, 0)),
        ),
        out_shape=(
            jax.ShapeDtypeStruct((K, CB, O), jnp.bfloat16),
            jax.ShapeDtypeStruct((H, K * O), jnp.bfloat16),
            jax.ShapeDtypeStruct((C, B, H), jnp.bfloat16),
            jax.ShapeDtypeStruct((B, H), jnp.float32),
        ),
        scratch_shapes=[
            pltpu.VMEM((H, H), jnp.bfloat16),
            pltpu.VMEM((I, H), jnp.bfloat16),
            pltpu.VMEM((1, H), jnp.float32),
            pltpu.VMEM((H, O), jnp.bfloat16),
            pltpu.VMEM((H, H), jnp.bfloat16),
            pltpu.VMEM((CB, H), jnp.float32),
        ],
        compiler_params=pltpu.CompilerParams(
            dimension_semantics=("arbitrary",)),
        cost_estimate=pl.CostEstimate(
            flops=2 * T * B * H * (H + I + O) + 8 * H * H * H,
            transcendentals=0,
            bytes_accessed=(T * B * I * 4 + T * B * O * 2 + C * B * H * 2
                            + B * H * 4)),
    )(xs4, whx, bhx, whh, bhh, woh, h0)

    y4 = pl.pallas_call(
        functools.partial(_head_kernel, cb=CB),
        grid=(K,),
        in_specs=[
            pl.BlockSpec((1, CB, O), lambda j: (j, 0, 0)),
            pl.BlockSpec((C, B, H), lambda j: (0, 0, 0)),
            pl.BlockSpec((H, O), lambda j: (0, j)),
            pl.BlockSpec((1, O), lambda j: (0, 0)),
        ],
        out_specs=pl.BlockSpec((C, 1, B, O), lambda j: (0, j, 0, 0)),
        out_shape=jax.ShapeDtypeStruct((C, K, B, O), jnp.float32),
        compiler_params=pltpu.CompilerParams(
            dimension_semantics=("arbitrary",)),
        cost_estimate=pl.CostEstimate(
            flops=2 * T * B * H * O, transcendentals=T * B * (O + 1),
            bytes_accessed=T * B * O * 2 + C * B * H * 2 + T * B * O * 4),
    )(l_all, s_prev, z, boh)

    return y4.reshape(T, B, O), h_final


# speedup vs baseline: 1.1261x; 1.1261x over previous
import functools

import jax
import jax.numpy as jnp
from jax.experimental import pallas as pl
from jax.experimental.pallas import tpu as pltpu


_K = 8


def _f32dot(a, b):
    return jnp.dot(a, b, preferred_element_type=jnp.float32)


def _bf16dot(a, b):
    return _f32dot(a, b).astype(jnp.bfloat16)


def _fused_kernel(x_ref, whx_ref, bhx_ref, whh_ref, bhh_ref, woh_ref,
                  boh_ref, h0_ref,
                  y_ref, hfin_ref,
                  wbf, wxp, bp, wohbf, wkp, zsc, carry, lsc,
                  *, k_steps, n_chunks, b):
    j = pl.program_id(0)
    cb = n_chunks * b

    @pl.when(j == 0)
    def _prep():
        w = whh_ref[...].astype(jnp.bfloat16)
        wbf[...] = w
        wxp[...] = _bf16dot(whx_ref[...].astype(jnp.bfloat16), w)
        bp[...] = _f32dot(bhx_ref[...].astype(jnp.bfloat16), w) + bhh_ref[...]
        woh_bf = woh_ref[...].astype(jnp.bfloat16)
        wohbf[...] = woh_bf
        w2 = _bf16dot(w, w)
        w4 = _bf16dot(w2, w2)
        wkp[...] = _bf16dot(w4, w4)
        z1 = _bf16dot(w, woh_bf)
        z2 = _bf16dot(w2, woh_bf)
        z12 = jnp.concatenate([z1, z2], axis=1)
        z34 = _bf16dot(w2, z12)
        z58 = _bf16dot(w4, jnp.concatenate([z12, z34], axis=1))
        o = woh_bf.shape[1]
        zsc[0] = z1
        zsc[1] = z2
        for i in range(2):
            zsc[2 + i] = z34[:, i * o:(i + 1) * o]
        for i in range(4):
            zsc[4 + i] = z58[:, i * o:(i + 1) * o]
        carry[...] = jnp.zeros_like(carry)
        carry[0:b, :] = h0_ref[...]

    @pl.when(j < k_steps)
    def _scan_step():
        v = _f32dot(x_ref[...].reshape(cb, -1).astype(jnp.bfloat16),
                    wxp[...]) + bp[...]
        r = _f32dot(carry[...].astype(jnp.bfloat16), wbf[...]) + v
        carry[...] = r
        lsc[j] = _bf16dot(r.astype(jnp.bfloat16), wohbf[...])

    @pl.when(j == k_steps)
    def _boundary():
        wk = wkp[...]
        s = carry[0:b, :]
        carry[0:b, :] = jnp.zeros_like(s)
        for c in range(1, n_chunks):
            e = carry[c * b:(c + 1) * b, :]
            carry[c * b:(c + 1) * b, :] = s
            s = _f32dot(s.astype(jnp.bfloat16), wk) + e
        hfin_ref[...] = s

    @pl.when(j > k_steps)
    def _head():
        jj = j - (k_steps + 1)
        sp = carry[...].astype(jnp.bfloat16)
        logits = (lsc[jj].astype(jnp.float32)
                  + _f32dot(sp, zsc[jj]) + boh_ref[...])
        m = jnp.max(logits, axis=1, keepdims=True)
        sh = logits - m
        lse = jnp.log(jnp.sum(jnp.exp(sh), axis=1, keepdims=True))
        y_ref[...] = (sh - lse).reshape(y_ref.shape)


def kernel(xs, h0, whx, bhx, whh, bhh, woh, boh):
    T, B, I = xs.shape
    H = whh.shape[0]
    O = woh.shape[1]
    K = _K
    assert T % K == 0, (T, K)
    C = T // K
    CB = C * B

    xs4 = xs.reshape(C, K, B, I)
    y4, h_final = pl.pallas_call(
        functools.partial(_fused_kernel, k_steps=K, n_chunks=C, b=B),
        grid=(2 * K + 1,),
        in_specs=[
            pl.BlockSpec((C, 1, B, I),
                         lambda j: (0, jnp.minimum(j, K - 1), 0, 0)),
            pl.BlockSpec((I, H), lambda j: (0, 0)),
            pl.BlockSpec((1, H), lambda j: (0, 0)),
            pl.BlockSpec((H, H), lambda j: (0, 0)),
            pl.BlockSpec((1, H), lambda j: (0, 0)),
            pl.BlockSpec((H, O), lambda j: (0, 0)),
            pl.BlockSpec((1, O), lambda j: (0, 0)),
            pl.BlockSpec((B, H), lambda j: (0, 0)),
        ],
        out_specs=(
            pl.BlockSpec((C, 1, B, O),
                         lambda j: (0, jnp.maximum(j - (K + 1), 0), 0, 0)),
            pl.BlockSpec((B, H), lambda j: (0, 0)),
        ),
        out_shape=(
            jax.ShapeDtypeStruct((C, K, B, O), jnp.float32),
            jax.ShapeDtypeStruct((B, H), jnp.float32),
        ),
        scratch_shapes=[
            pltpu.VMEM((H, H), jnp.bfloat16),
            pltpu.VMEM((I, H), jnp.bfloat16),
            pltpu.VMEM((1, H), jnp.float32),
            pltpu.VMEM((H, O), jnp.bfloat16),
            pltpu.VMEM((H, H), jnp.bfloat16),
            pltpu.VMEM((K, H, O), jnp.bfloat16),
            pltpu.VMEM((CB, H), jnp.float32),
            pltpu.VMEM((K, CB, O), jnp.bfloat16),
        ],
        compiler_params=pltpu.CompilerParams(
            dimension_semantics=("arbitrary",)),
        cost_estimate=pl.CostEstimate(
            flops=2 * T * B * H * (H + I + 2 * O) + 8 * H * H * H,
            transcendentals=T * B * (O + 1),
            bytes_accessed=T * B * I * 4 + T * B * O * 4 + B * H * 4),
    )(xs4, whx, bhx, whh, bhh, woh, boh, h0)

    return y4.reshape(T, B, O), h_final
